# Initial kernel scaffold; baseline (speedup 1.0000x reference)
import jax
import jax.numpy as jnp
from jax.experimental import pallas as pl


def kernel(x, edge_index, batch, W1, a_src1, a_dst1, b1, W2, a_src2, a_dst2, b2, W3, a_src3, a_dst3, b3, cW1, cb1, cW2, cb2):
    raise NotImplementedError("write your pallas kernel here")


# speedup vs baseline: 16.7376x; 16.7376x over previous
import functools
import jax
import jax.numpy as jnp
from jax import lax
from jax.experimental import pallas as pl
from jax.experimental.pallas import tpu as pltpu
from jax.experimental.pallas import tpu_sc as plsc

_N = 10000
_D = 128
_HID = 64
_G = 64
_E_RAW = 320000
_E_TOT = _E_RAW + _N
_C = 128
_E_PAD = 331776
_E_HALF = _E_PAD // 2
_NACC = 10240
_ROWS_T = _NACC // 16
_W = 80
_BM = 400
_EPS = 1e-16

_SC_PARAMS = pltpu.CompilerParams(
    needs_layout_passes=False, use_tc_tiling_on_sc=False)


def _edge_body(split_edges, src_hbm, dst_hbm, h_hbm, as_hbm, ad_hbm, out_hbm,
               as_ref, ad_ref, hrows, msg, src_loc, src_gl, dst_loc,
               accum, sem):
    c = lax.axis_index("c")
    s = lax.axis_index("s")
    zero16 = jnp.zeros((16,), jnp.float32)

    if split_edges:
        pltpu.sync_copy(as_hbm, as_ref)
        pltpu.sync_copy(ad_hbm, ad_ref)
    else:
        pltpu.sync_copy(as_hbm.at[c], as_ref)
        pltpu.sync_copy(ad_hbm.at[c], ad_ref)

    def zrow(r, carry):
        for k in range(_W // 16):
            msg[r, pl.ds(k * 16, 16)] = zero16
        return carry

    lax.fori_loop(0, _C, zrow, 0)
    row0 = s * _ROWS_T
    for p in range(_ROWS_T // _C):
        pltpu.sync_copy(msg, accum.at[pl.ds(row0 + p * _C, _C)])
    plsc.subcore_barrier()

    iota16 = lax.iota(jnp.int32, 16)
    per_tile = (_E_HALF if split_edges else _E_PAD) // 16
    nchunks = per_tile // _C
    ebase = c * _E_HALF + s * per_tile if split_edges else s * per_tile
    cn = 0 if split_edges else c * _N

    def chunk(j, carry):
        eoff = ebase + j * _C
        pltpu.sync_copy(src_hbm.at[pl.ds(eoff, _C)], src_loc)
        pltpu.sync_copy(dst_hbm.at[pl.ds(eoff, _C)], dst_loc)
        if split_edges:
            gather_idx = src_loc
        else:
            for g in range(8):
                sl = pl.ds(g * 16, 16)
                src_gl[sl] = src_loc[sl] + cn
            gather_idx = src_gl
        cp = pltpu.async_copy(h_hbm.at[gather_idx], hrows, sem)
        for g in range(8):
            sl = pl.ds(g * 16, 16)
            s16 = src_loc[sl]
            d16 = dst_loc[sl]
            asv = plsc.load_gather(as_ref, [s16])
            adv = plsc.load_gather(ad_ref, [d16])
            e = asv + adv
            e = jnp.maximum(e, 0.2 * e)
            w = jnp.exp(e)
            plsc.store_scatter(
                msg, [iota16 + g * 16, jnp.full((16,), 64, jnp.int32)], w)
        cp.wait()

        def scale(i, carry):
            wv = msg[i, pl.ds(64, 16)]
            w0 = jnp.broadcast_to(wv[0], (16,))
            for k in range(4):
                msg[i, pl.ds(k * 16, 16)] = (
                    hrows[i, pl.ds(k * 16, 16)] * w0)
            return carry

        lax.fori_loop(0, _C, scale, 0)
        pltpu.sync_copy(msg, accum.at[dst_loc], add=True)
        return carry

    lax.fori_loop(0, nchunks, chunk, 0)
    plsc.subcore_barrier()
    for p in range(_ROWS_T // _C):
        r0 = row0 + p * _C
        pltpu.sync_copy(accum.at[pl.ds(r0, _C)], msg)
        pltpu.sync_copy(msg, out_hbm.at[pl.ds(c * _NACC + r0, _C)])


def _edge_call(split_edges, src_pad, dst_pad, h_hbm, as_hbm, ad_hbm):
    mesh = plsc.VectorSubcoreMesh(core_axis_name="c", subcore_axis_name="s")
    kfn = pl.kernel(
        functools.partial(_edge_body, split_edges),
        out_type=jax.ShapeDtypeStruct((2 * _NACC, _W), jnp.float32),
        mesh=mesh,
        compiler_params=_SC_PARAMS,
        scratch_types=[
            pltpu.VMEM((_N,), jnp.float32),
            pltpu.VMEM((_N,), jnp.float32),
            pltpu.VMEM((_C, 64), jnp.float32),
            pltpu.VMEM((_C, _W), jnp.float32),
            pltpu.VMEM((_C,), jnp.int32),
            pltpu.VMEM((_C,), jnp.int32),
            pltpu.VMEM((_C,), jnp.int32),
            pltpu.VMEM_SHARED((_NACC, _W), jnp.float32),
            pltpu.SemaphoreType.DMA,
        ],
    )
    return kfn(src_pad, dst_pad, h_hbm, as_hbm, ad_hbm)


def _k1_body(x_ref, w_ref, asm_ref, adm_ref, h_ref, as_ref, ad_ref):
    h = jnp.dot(x_ref[...], w_ref[...], preferred_element_type=jnp.float32)
    h_ref[...] = h
    as_ref[...] = jnp.dot(h, asm_ref[...], preferred_element_type=jnp.float32,
                          precision=lax.Precision.HIGHEST)
    ad_ref[...] = jnp.dot(h, adm_ref[...], preferred_element_type=jnp.float32,
                          precision=lax.Precision.HIGHEST)


def _tc_layer1(x, w1, asm, adm):
    nb = _N // _BM
    return pl.pallas_call(
        _k1_body,
        grid=(nb,),
        in_specs=[
            pl.BlockSpec((_BM, _D), lambda i: (i, 0)),
            pl.BlockSpec((_D, 256), lambda i: (0, 0)),
            pl.BlockSpec((256, 8), lambda i: (0, 0)),
            pl.BlockSpec((256, 8), lambda i: (0, 0)),
        ],
        out_specs=[
            pl.BlockSpec((_BM, 256), lambda i: (i, 0)),
            pl.BlockSpec((_BM, 8), lambda i: (i, 0)),
            pl.BlockSpec((_BM, 8), lambda i: (i, 0)),
        ],
        out_shape=[
            jax.ShapeDtypeStruct((_N, 256), jnp.float32),
            jax.ShapeDtypeStruct((_N, 8), jnp.float32),
            jax.ShapeDtypeStruct((_N, 8), jnp.float32),
        ],
    )(x, w1, asm, adm)


def _k23_body(num_ref, den_ref, r_ref, b_ref, w_ref, asm_ref, adm_ref,
              h_ref, as_ref, ad_ref):
    rec = 1.0 / (den_ref[...] + _EPS)
    rep = jnp.dot(rec, r_ref[...], preferred_element_type=jnp.float32,
                  precision=lax.Precision.HIGHEST)
    a = num_ref[...] * rep + b_ref[...]
    act = jnp.where(a > 0, a, jnp.exp(a) - 1.0)
    h = jnp.dot(act, w_ref[...], preferred_element_type=jnp.float32)
    h_ref[...] = h
    as_ref[...] = jnp.dot(h, asm_ref[...], preferred_element_type=jnp.float32,
                          precision=lax.Precision.HIGHEST)
    ad_ref[...] = jnp.dot(h, adm_ref[...], preferred_element_type=jnp.float32,
                          precision=lax.Precision.HIGHEST)


def _tc_layer23(num, den, rmat, bias, w, asm, adm, out_w):
    nb = _N // _BM
    return pl.pallas_call(
        _k23_body,
        grid=(nb,),
        in_specs=[
            pl.BlockSpec((_BM, 256), lambda i: (i, 0)),
            pl.BlockSpec((_BM, 8), lambda i: (i, 0)),
            pl.BlockSpec((8, 256), lambda i: (0, 0)),
            pl.BlockSpec((1, 256), lambda i: (0, 0)),
            pl.BlockSpec((256, out_w), lambda i: (0, 0)),
            pl.BlockSpec((out_w, 8), lambda i: (0, 0)),
            pl.BlockSpec((out_w, 8), lambda i: (0, 0)),
        ],
        out_specs=[
            pl.BlockSpec((_BM, out_w), lambda i: (i, 0)),
            pl.BlockSpec((_BM, 8), lambda i: (i, 0)),
            pl.BlockSpec((_BM, 8), lambda i: (i, 0)),
        ],
        out_shape=[
            jax.ShapeDtypeStruct((_N, out_w), jnp.float32),
            jax.ShapeDtypeStruct((_N, 8), jnp.float32),
            jax.ShapeDtypeStruct((_N, 8), jnp.float32),
        ],
    )(num, den, rmat, bias, w, asm, adm)


def _k4_body(a_ref, b_ref, bat_ref, b3_ref, cw1_ref, cb1_ref, cw2_ref,
             cb2_ref, out_ref, acc_ref):
    i = pl.program_id(0)

    @pl.when(i == 0)
    def _():
        acc_ref[...] = jnp.zeros_like(acc_ref)

    a = a_ref[...]
    b = b_ref[...]
    num = a[:, :64] + b[:, :64]
    den = a[:, 64:65] + b[:, 64:65]
    h = num / (den + _EPS) + b3_ref[...]
    bat = bat_ref[...]
    g64 = lax.broadcasted_iota(jnp.int32, (1, _G), 1)
    mask = (bat == g64).astype(jnp.float32)
    feat = jnp.concatenate([h, jnp.ones_like(h)], axis=1)
    acc_ref[...] += lax.dot_general(
        mask, feat, (((0,), (0,)), ((), ())),
        preferred_element_type=jnp.float32,
        precision=lax.Precision.HIGHEST)

    @pl.when(i == (_N // _BM) - 1)
    def _():
        gf = acc_ref[...]
        g = gf[:, :64] / jnp.maximum(gf[:, 64:65], 1.0)
        z = jnp.dot(g, cw1_ref[...], preferred_element_type=jnp.float32,
                    precision=lax.Precision.HIGHEST)
        z = jnp.maximum(z + cb1_ref[...], 0.0)
        out_ref[...] = jnp.dot(
            z, cw2_ref[...], preferred_element_type=jnp.float32,
            precision=lax.Precision.HIGHEST) + cb2_ref[...]


def _tc_final(acc_a, acc_b, batch2d, b3, cw1, cb1, cw2p, cb2p):
    nb = _N // _BM
    return pl.pallas_call(
        _k4_body,
        grid=(nb,),
        in_specs=[
            pl.BlockSpec((_BM, _W), lambda i: (i, 0)),
            pl.BlockSpec((_BM, _W), lambda i: (i, 0)),
            pl.BlockSpec((_BM, 1), lambda i: (i, 0)),
            pl.BlockSpec((1, 64), lambda i: (0, 0)),
            pl.BlockSpec((64, 32), lambda i: (0, 0)),
            pl.BlockSpec((1, 32), lambda i: (0, 0)),
            pl.BlockSpec((32, 8), lambda i: (0, 0)),
            pl.BlockSpec((1, 8), lambda i: (0, 0)),
        ],
        out_specs=pl.BlockSpec((_G, 8), lambda i: (0, 0)),
        out_shape=jax.ShapeDtypeStruct((_G, 8), jnp.float32),
        scratch_shapes=[pltpu.VMEM((_G, 128), jnp.float32)],
    )(acc_a, acc_b, batch2d, b3, cw1, cb1, cw2p, cb2p)


def _blockdiag(a):
    hh = a.shape[0]
    eye = jnp.eye(hh, 8, dtype=a.dtype)
    return (a[:, :, None] * eye[:, None, :]).reshape(hh * _HID, 8)


def _four_head_edges(src_pad, dst_pad, h, as4, ad4):
    nums = []
    dens = []
    prev = None
    for pair in range(2):
        h_stk = jnp.concatenate(
            [h[:, (2 * pair) * 64:(2 * pair + 1) * 64],
             h[:, (2 * pair + 1) * 64:(2 * pair + 2) * 64]], axis=0)
        as_stk = jnp.stack([as4[:, 2 * pair], as4[:, 2 * pair + 1]])
        ad_stk = jnp.stack([ad4[:, 2 * pair], ad4[:, 2 * pair + 1]])
        if prev is not None:
            h_stk, _ = lax.optimization_barrier((h_stk, prev))
        acc = _edge_call(False, src_pad, dst_pad, h_stk, as_stk, ad_stk)
        prev = acc
        nums.append(acc[:_N, :64])
        nums.append(acc[_NACC:_NACC + _N, :64])
        dens.append(acc[:_N, 64:65])
        dens.append(acc[_NACC:_NACC + _N, 64:65])
    num = jnp.concatenate(nums, axis=1)
    den = jnp.concatenate(
        dens + [jnp.ones((_N, 4), jnp.float32)], axis=1)
    return num, den


def kernel(x, edge_index, batch, W1, a_src1, a_dst1, b1, W2, a_src2, a_dst2,
           b2, W3, a_src3, a_dst3, b3, cW1, cb1, cW2, cb2):
    loops = jnp.arange(_N, dtype=jnp.int32)
    src = jnp.concatenate([edge_index[0], loops])
    dst = jnp.concatenate([edge_index[1], loops])
    npad = _E_PAD - _E_TOT
    src_pad = jnp.concatenate([src, jnp.zeros((npad,), jnp.int32)])
    dst_pad = jnp.concatenate([dst, jnp.full((npad,), _N, jnp.int32)])

    asm1 = _blockdiag(a_src1)
    adm1 = _blockdiag(a_dst1)
    asm2 = _blockdiag(a_src2)
    adm2 = _blockdiag(a_dst2)
    asm3 = jnp.concatenate(
        [a_src3.T, jnp.zeros((_HID, 7), jnp.float32)], axis=1)
    adm3 = jnp.concatenate(
        [a_dst3.T, jnp.zeros((_HID, 7), jnp.float32)], axis=1)
    rmat = jnp.concatenate(
        [jnp.repeat(jnp.eye(4, dtype=jnp.float32), 64, axis=1),
         jnp.zeros((4, 256), jnp.float32)], axis=0)

    h1, as1, ad1 = _tc_layer1(x, W1, asm1, adm1)
    num1, den1 = _four_head_edges(src_pad, dst_pad, h1, as1, ad1)

    h2, as2, ad2 = _tc_layer23(num1, den1, rmat, b1.reshape(1, 256), W2,
                               asm2, adm2, 256)
    num2, den2 = _four_head_edges(src_pad, dst_pad, h2, as2, ad2)

    h3, as3, ad3 = _tc_layer23(num2, den2, rmat, b2.reshape(1, 256), W3,
                               asm3, adm3, 64)
    acc3 = _edge_call(True, src_pad, dst_pad, h3, as3[:, 0], ad3[:, 0])

    out = _tc_final(
        acc3[:_N], acc3[_NACC:_NACC + _N], batch.reshape(_N, 1),
        b3.reshape(1, 64), cW1, cb1.reshape(1, 32),
        jnp.concatenate([cW2, jnp.zeros((32, 7), jnp.float32)], axis=1),
        jnp.concatenate([cb2, jnp.zeros((7,), jnp.float32)]).reshape(1, 8))
    return out[:, :1]
